# baseline (speedup 1.0000x reference)
import functools

import jax
import jax.numpy as jnp
from jax import lax
from jax.experimental import pallas as pl
from jax.experimental.pallas import tpu as pltpu
from jax.experimental.pallas import tpu_sc as plsc

N = 1024
C = 100000
L = 16
NS = 16
PER = N // NS


def _sc_body(prob_hbm, target_hbm, reward_hbm, out_hbm,
             tgt_v, rwd_v, idx_v, val_v, part_v, acc_v, out_v, shared, sem):
    sid = lax.axis_index("s")
    base = sid * PER

    pltpu.sync_copy(target_hbm.at[pl.ds(base, PER)], tgt_v)
    pltpu.sync_copy(reward_hbm.at[pl.ds(base, PER)], rwd_v)

    lane = lax.iota(jnp.int32, L)
    for j in range(PER // L):
        row0 = base + j * L
        idx_v[pl.ds(j * L, L)] = tgt_v[pl.ds(j * L, L)] + (row0 + lane) * C

    pltpu.async_copy(prob_hbm.at[idx_v], val_v, sem).wait()

    acc = jnp.zeros((L,), jnp.float32)
    for j in range(PER // L):
        acc = acc + val_v[pl.ds(j * L, L)] * rwd_v[pl.ds(j * L, L)]
    part_v[...] = acc

    pltpu.sync_copy(part_v, shared.at[pl.ds(sid * L, L)])
    plsc.subcore_barrier()

    @pl.when(sid == 0)
    def _():
        pltpu.sync_copy(shared, acc_v)
        tot = jnp.zeros((L,), jnp.float32)
        for i in range(NS):
            tot = tot + acc_v[pl.ds(i * L, L)]
        s = tot[0]
        for i in range(1, L):
            s = s + tot[i]
        out_v[...] = jnp.broadcast_to(-s, (L,))
        pltpu.sync_copy(out_v, out_hbm)


@jax.jit
def _sc_loss(prob_flat, target, reward):
    mesh = plsc.VectorSubcoreMesh(
        core_axis_name="c", subcore_axis_name="s", num_cores=1, num_subcores=NS)
    run = pl.kernel(
        _sc_body,
        out_type=jax.ShapeDtypeStruct((L,), jnp.float32),
        mesh=mesh,
        scratch_types=[
            pltpu.VMEM((PER,), jnp.int32),
            pltpu.VMEM((PER,), jnp.float32),
            pltpu.VMEM((PER,), jnp.int32),
            pltpu.VMEM((PER,), jnp.float32),
            pltpu.VMEM((L,), jnp.float32),
            pltpu.VMEM((NS * L,), jnp.float32),
            pltpu.VMEM((L,), jnp.float32),
            pltpu.VMEM_SHARED((NS * L,), jnp.float32),
            pltpu.SemaphoreType.DMA,
        ],
    )
    return run(prob_flat, target, reward)


def kernel(prob, target, reward):
    out = _sc_loss(prob.reshape(-1), target.astype(jnp.int32), reward)
    return out[0]


# speedup vs baseline: 2.3574x; 2.3574x over previous
import jax
import jax.numpy as jnp
from jax import lax
from jax.experimental import pallas as pl
from jax.experimental.pallas import tpu as pltpu
from jax.experimental.pallas import tpu_sc as plsc

N = 1024
C = 100000
L = 16
NS = 16
PER = N // NS
W = 16


def _sc_body(prob_hbm, target_hbm, reward_hbm, out_hbm,
             tgt_v, rwd_v, win_v, part_v, acc_v, out_v, shared, sem):
    sid = lax.axis_index("s")
    base = sid * PER

    pltpu.sync_copy(target_hbm.at[pl.ds(base, PER)], tgt_v)
    pltpu.sync_copy(reward_hbm.at[pl.ds(base, PER)], rwd_v)

    tscal, rscal = [], []
    for j in range(PER // L):
        tv = tgt_v[pl.ds(j * L, L)]
        rv = rwd_v[pl.ds(j * L, L)]
        for i in range(L):
            tscal.append(tv[i])
            rscal.append(rv[i])

    copies = []
    for r in range(PER):
        c0 = (tscal[r] // W) * W
        copies.append(pltpu.async_copy(
            prob_hbm.at[base + r, pl.ds(c0, W)],
            win_v.at[pl.ds(r * W, W)], sem))
    for c in copies:
        c.wait()

    lane = lax.iota(jnp.int32, L)
    acc = jnp.zeros((L,), jnp.float32)
    for r in range(PER):
        wv = win_v[pl.ds(r * W, W)]
        acc = acc + jnp.where(lane == tscal[r] % W, wv, 0.0) * rscal[r]
    part_v[...] = acc

    pltpu.sync_copy(part_v, shared.at[pl.ds(sid * L, L)])
    plsc.subcore_barrier()

    @pl.when(sid == 0)
    def _():
        pltpu.sync_copy(shared, acc_v)
        tot = jnp.zeros((L,), jnp.float32)
        for i in range(NS):
            tot = tot + acc_v[pl.ds(i * L, L)]
        s = tot[0]
        for i in range(1, L):
            s = s + tot[i]
        out_v[...] = jnp.broadcast_to(-s, (L,))
        pltpu.sync_copy(out_v, out_hbm)


@jax.jit
def _sc_loss(prob, target, reward):
    mesh = plsc.VectorSubcoreMesh(
        core_axis_name="c", subcore_axis_name="s", num_cores=1, num_subcores=NS)
    run = pl.kernel(
        _sc_body,
        out_type=jax.ShapeDtypeStruct((L,), jnp.float32),
        mesh=mesh,
        scratch_types=[
            pltpu.VMEM((PER,), jnp.int32),
            pltpu.VMEM((PER,), jnp.float32),
            pltpu.VMEM((PER * W,), jnp.float32),
            pltpu.VMEM((L,), jnp.float32),
            pltpu.VMEM((NS * L,), jnp.float32),
            pltpu.VMEM((L,), jnp.float32),
            pltpu.VMEM_SHARED((NS * L,), jnp.float32),
            pltpu.SemaphoreType.DMA,
        ],
    )
    return run(prob, target, reward)


def kernel(prob, target, reward):
    out = _sc_loss(prob, target.astype(jnp.int32), reward)
    return out[0]


# speedup vs baseline: 41.8387x; 17.7481x over previous
import jax
import jax.numpy as jnp
from jax import lax
from jax.experimental import pallas as pl
from jax.experimental.pallas import tpu as pltpu
from jax.experimental.pallas import tpu_sc as plsc

N = 1024
C = 100000
L = 16
NS = 16
PER = N // NS


def _sc_body(probt_hbm, target_hbm, reward_hbm, out_hbm,
             tgt_v, rwd_v, win_v, part_v, acc_v, out_v, shared, sem):
    sid = lax.axis_index("s")
    base = sid * PER

    pltpu.sync_copy(target_hbm.at[pl.ds(base, PER)], tgt_v)
    pltpu.sync_copy(reward_hbm.at[pl.ds(base, PER)], rwd_v)

    c0 = pl.multiple_of((sid // 2) * 128, 128)
    off = (sid % 2) * PER
    pltpu.async_copy(
        probt_hbm.at[tgt_v, pl.ds(c0, 2 * PER)], win_v, sem).wait()

    lane = lax.iota(jnp.int32, L)
    acc = jnp.zeros((L,), jnp.float32)
    for j in range(PER // L):
        rv = rwd_v[pl.ds(j * L, L)]
        for k in range(L):
            r = j * L + k
            wv = win_v[r, pl.ds(off + j * L, L)]
            acc = acc + jnp.where(lane == k, wv * rv[k], 0.0)
    part_v[...] = acc

    pltpu.sync_copy(part_v, shared.at[pl.ds(sid * L, L)])
    plsc.subcore_barrier()

    @pl.when(sid == 0)
    def _():
        pltpu.sync_copy(shared, acc_v)
        tot = jnp.zeros((L,), jnp.float32)
        for i in range(NS):
            tot = tot + acc_v[pl.ds(i * L, L)]
        s = tot[0]
        for i in range(1, L):
            s = s + tot[i]
        out_v[...] = jnp.broadcast_to(-s, (L,))
        pltpu.sync_copy(out_v, out_hbm)


@jax.jit
def _sc_loss(probt, target, reward):
    mesh = plsc.VectorSubcoreMesh(
        core_axis_name="c", subcore_axis_name="s", num_cores=1, num_subcores=NS)
    run = pl.kernel(
        _sc_body,
        out_type=jax.ShapeDtypeStruct((L,), jnp.float32),
        mesh=mesh,
        scratch_types=[
            pltpu.VMEM((PER,), jnp.int32),
            pltpu.VMEM((PER,), jnp.float32),
            pltpu.VMEM((PER, 2 * PER), jnp.float32),
            pltpu.VMEM((L,), jnp.float32),
            pltpu.VMEM((NS * L,), jnp.float32),
            pltpu.VMEM((L,), jnp.float32),
            pltpu.VMEM_SHARED((NS * L,), jnp.float32),
            pltpu.SemaphoreType.DMA,
        ],
    )
    return run(probt, target, reward)


def kernel(prob, target, reward):
    out = _sc_loss(prob.T, target.astype(jnp.int32), reward)
    return out[0]
